# baseline (speedup 1.0000x reference)
import jax
import jax.numpy as jnp
from jax import lax
from jax.experimental import pallas as pl
from jax.experimental.pallas import tpu as pltpu
from jax.experimental.pallas import tpu_sc as plsc

_N = 131072
_NB = 36
_C = 8
_H = 65536
_V = _NB * _H
_OUT = 3 + _NB * _C

_NW = 32
_NPW = _N // _NW
_CH = 128
_NCH = _NPW // _CH


_PNB = 2048


def _prep_body(scales_ref, pts_ref, idx_ref, w1_ref, y_ref):
    p3 = pts_ref[...]
    encs = []
    for f in range(6):
        s = scales_ref[f]
        fp = p3 * s
        encs.append(jnp.sin(fp))
        encs.append(jnp.cos(fp))
    enc = jnp.concatenate(encs, axis=0)
    y_ref[...] = enc
    iy = (enc + 1.0) * 32767.5
    iy0 = jnp.floor(iy)
    w1_ref[...] = iy - iy0
    i0 = jnp.clip(iy0, 0.0, 65535.0).astype(jnp.int32)
    boff = lax.broadcasted_iota(jnp.int32, (_NB, _PNB), 0) * _H
    g0 = i0 + boff
    g1 = jnp.minimum(g0 + 1, _V - 1)
    idx_ref[0, :, :] = g0
    idx_ref[1, :, :] = g1


def _prep(scales, points_t):
    return pl.pallas_call(
        _prep_body,
        grid=(_N // _PNB,),
        in_specs=[
            pl.BlockSpec(memory_space=pltpu.SMEM),
            pl.BlockSpec((3, _PNB), lambda i: (0, i)),
        ],
        out_specs=[
            pl.BlockSpec((2, _NB, _PNB), lambda i: (0, 0, i)),
            pl.BlockSpec((_NB, _PNB), lambda i: (0, i)),
            pl.BlockSpec((_NB, _PNB), lambda i: (0, i)),
        ],
        out_shape=[
            jax.ShapeDtypeStruct((2, _NB, _N), jnp.int32),
            jax.ShapeDtypeStruct((_NB, _N), jnp.float32),
            jax.ShapeDtypeStruct((_NB, _N), jnp.float32),
        ],
    )(scales, points_t)


_TWB = 2048


def _tr_body(cv_ref, t_ref):
    t_ref[...] = cv_ref[0].T


def _table(cv):
    return pl.pallas_call(
        _tr_body,
        grid=(_NB, _H // _TWB),
        in_specs=[pl.BlockSpec((1, _C, _TWB), lambda b, j: (b, 0, j))],
        out_specs=pl.BlockSpec((_TWB, _C), lambda b, j: (b * (_H // _TWB) + j, 0)),
        out_shape=jax.ShapeDtypeStruct((_V, _C), jnp.float32),
    )(cv)


def _sc_body(table, idxI, w1T, yT, pts, out, idxb, w1b, yb, ptsb, R0, R1, O, gsem):
    wid = lax.axis_index("s") * 2 + lax.axis_index("c")
    base = wid * _NPW
    iota = lax.iota(jnp.int32, 16)

    @pl.loop(0, _NCH)
    def _chunk(i):
        n0 = base + i * _CH
        pltpu.sync_copy(idxI.at[:, :, pl.ds(n0, _CH)], idxb)
        pltpu.sync_copy(w1T.at[:, pl.ds(n0, _CH)], w1b)
        pltpu.sync_copy(yT.at[:, pl.ds(n0, _CH)], yb)
        pltpu.sync_copy(pts.at[pl.ds(n0, _CH), :], ptsb)

        @pl.loop(0, _NB)
        def _fire(b):
            pltpu.async_copy(table.at[idxb.at[0, b]], R0.at[b], gsem)
            pltpu.async_copy(table.at[idxb.at[1, b]], R1.at[b], gsem)

        @pl.loop(0, _NB)
        def _drain(b):
            pltpu.make_async_copy(table.at[idxb.at[0, b]], R0.at[b], gsem).wait()
            pltpu.make_async_copy(table.at[idxb.at[1, b]], R1.at[b], gsem).wait()

        for q in range(_CH // 16):
            prow = iota + q * 16
            for d in range(3):
                dd = jnp.full((16,), d, jnp.int32)
                v = plsc.load_gather(ptsb, [prow, dd])
                plsc.store_scatter(O, [prow, dd], v)

            @pl.loop(0, _NB)
            def _b(b):
                w1v = w1b[b, pl.ds(q * 16, 16)]
                yv = yb[b, pl.ds(q * 16, 16)]
                bs = jnp.full((16,), b, jnp.int32)
                for c in range(_C):
                    cs = jnp.full((16,), c, jnp.int32)
                    v0 = plsc.load_gather(R0, [bs, prow, cs])
                    v1 = plsc.load_gather(R1, [bs, prow, cs])
                    o = v0 + w1v * (v1 - v0) + yv
                    col = jnp.full((16,), 3 + c * _NB, jnp.int32) + bs
                    plsc.store_scatter(O, [prow, col], o)

        pltpu.sync_copy(O, out.at[pl.ds(n0, _CH), :])


def _sc(table, idxI, w1T, yT, pts):
    mesh = plsc.VectorSubcoreMesh(
        core_axis_name="c", subcore_axis_name="s", num_cores=2, num_subcores=16
    )
    return pl.kernel(
        _sc_body,
        out_type=jax.ShapeDtypeStruct((_N, _OUT), jnp.float32),
        mesh=mesh,
        compiler_params=pltpu.CompilerParams(
            needs_layout_passes=False, use_tc_tiling_on_sc=False
        ),
        scratch_types=[
            pltpu.VMEM((2, _NB, _CH), jnp.int32),
            pltpu.VMEM((_NB, _CH), jnp.float32),
            pltpu.VMEM((_NB, _CH), jnp.float32),
            pltpu.VMEM((_CH, 3), jnp.float32),
            pltpu.VMEM((_NB, _CH, _C), jnp.float32),
            pltpu.VMEM((_NB, _CH, _C), jnp.float32),
            pltpu.VMEM((_CH, _OUT), jnp.float32),
            pltpu.SemaphoreType.DMA,
        ],
    )(table, idxI, w1T, yT, pts)


def kernel(points, features, scales):
    cv = features[..., 0]
    table = _table(cv)
    idxI, w1T, yT = _prep(scales, points.T)
    return _sc(table, idxI, w1T, yT, points)


# speedup vs baseline: 1.7077x; 1.7077x over previous
import jax
import jax.numpy as jnp
from jax import lax
from jax.experimental import pallas as pl
from jax.experimental.pallas import tpu as pltpu
from jax.experimental.pallas import tpu_sc as plsc

_N = 131072
_NB = 36
_C = 8
_H = 65536
_V = _NB * _H
_OUT = 3 + _NB * _C

_NW = 32
_NPW = _N // _NW
_CH = 128
_NCH = _NPW // _CH


_PNB = 2048


def _prep_body(scales_ref, pts_ref, idx_ref, w1_ref, y_ref):
    p3 = pts_ref[...]
    encs = []
    for f in range(6):
        s = scales_ref[f]
        fp = p3 * s
        encs.append(jnp.sin(fp))
        encs.append(jnp.cos(fp))
    enc = jnp.concatenate(encs, axis=0)
    y_ref[...] = enc
    iy = (enc + 1.0) * 32767.5
    iy0 = jnp.floor(iy)
    w1_ref[...] = iy - iy0
    i0 = jnp.clip(iy0, 0.0, 65535.0).astype(jnp.int32)
    boff = lax.broadcasted_iota(jnp.int32, (_NB, _PNB), 0) * _H
    g0 = i0 + boff
    g1 = jnp.minimum(g0 + 1, _V - 1)
    idx_ref[0, :, :] = g0
    idx_ref[1, :, :] = g1


def _prep(scales, points_t):
    return pl.pallas_call(
        _prep_body,
        grid=(_N // _PNB,),
        in_specs=[
            pl.BlockSpec(memory_space=pltpu.SMEM),
            pl.BlockSpec((3, _PNB), lambda i: (0, i)),
        ],
        out_specs=[
            pl.BlockSpec((2, _NB, _PNB), lambda i: (0, 0, i)),
            pl.BlockSpec((_NB, _PNB), lambda i: (0, i)),
            pl.BlockSpec((_NB, _PNB), lambda i: (0, i)),
        ],
        out_shape=[
            jax.ShapeDtypeStruct((2, _NB, _N), jnp.int32),
            jax.ShapeDtypeStruct((_NB, _N), jnp.float32),
            jax.ShapeDtypeStruct((_NB, _N), jnp.float32),
        ],
    )(scales, points_t)


_TW = 2048
_NU = _NB * (_H // _TW)
_UPW = _NU // _NW


_NWIN = _H // _TW


def _tr_body(cv, table, xb, ob, isem, osem):
    wid = lax.axis_index("s") * 2 + lax.axis_index("c")
    iota = lax.iota(jnp.int32, 16)
    u0 = wid * _UPW
    cvec = jnp.bitwise_and(iota, 7)
    pvec = lax.shift_right_logical(iota, 3)

    def src(u):
        return cv.at[u // _NWIN, :, pl.ds((u % _NWIN) * _TW, _TW)]

    def dst(u):
        return table.at[pl.ds((u // _NWIN) * _H + (u % _NWIN) * _TW, _TW), :]

    pltpu.async_copy(src(u0), xb.at[0], isem.at[0])

    @pl.loop(0, _UPW)
    def _unit(j):
        u = u0 + j
        slot = lax.rem(j, 2)
        pltpu.make_async_copy(src(u), xb.at[slot], isem.at[slot]).wait()

        @pl.when(j + 1 < _UPW)
        def _next():
            pltpu.async_copy(src(u + 1), xb.at[1 - slot], isem.at[1 - slot])

        @pl.when(j >= 2)
        def _wo():
            pltpu.make_async_copy(ob.at[slot], dst(u), osem.at[slot]).wait()

        xf = xb.at[slot]
        of = ob.at[slot]

        @pl.loop(0, _TW // 2, unroll=4)
        def _col(p):
            col = pvec + p * 2
            v = plsc.load_gather(xf, [cvec, col])
            plsc.store_scatter(of, [col, cvec], v)

        pltpu.async_copy(ob.at[slot], dst(u), osem.at[slot])

    pltpu.make_async_copy(ob.at[0], dst(u0), osem.at[0]).wait()
    pltpu.make_async_copy(ob.at[1], dst(u0), osem.at[1]).wait()


def _table(cv):
    mesh = plsc.VectorSubcoreMesh(
        core_axis_name="c", subcore_axis_name="s", num_cores=2, num_subcores=16
    )
    return pl.kernel(
        _tr_body,
        out_type=jax.ShapeDtypeStruct((_V, _C), jnp.float32),
        mesh=mesh,
        compiler_params=pltpu.CompilerParams(
            needs_layout_passes=False, use_tc_tiling_on_sc=False
        ),
        scratch_types=[
            pltpu.VMEM((2, _C, _TW), jnp.float32),
            pltpu.VMEM((2, _TW, _C), jnp.float32),
            pltpu.SemaphoreType.DMA((2,)),
            pltpu.SemaphoreType.DMA((2,)),
        ],
    )(cv)


def _sc_body(table, idxI, w1T, yT, pts, out, idxb, w1b, yb, ptsb, R0, R1, O, gsem):
    wid = lax.axis_index("s") * 2 + lax.axis_index("c")
    base = wid * _NPW
    iota = lax.iota(jnp.int32, 16)

    @pl.loop(0, _NCH)
    def _chunk(i):
        n0 = base + i * _CH
        pltpu.sync_copy(idxI.at[:, :, pl.ds(n0, _CH)], idxb)
        pltpu.sync_copy(w1T.at[:, pl.ds(n0, _CH)], w1b)
        pltpu.sync_copy(yT.at[:, pl.ds(n0, _CH)], yb)
        pltpu.sync_copy(pts.at[pl.ds(n0, _CH), :], ptsb)

        @pl.loop(0, _NB)
        def _fire(b):
            pltpu.async_copy(table.at[idxb.at[0, b]], R0.at[b], gsem)
            pltpu.async_copy(table.at[idxb.at[1, b]], R1.at[b], gsem)

        @pl.loop(0, _NB)
        def _drain(b):
            pltpu.make_async_copy(table.at[idxb.at[0, b]], R0.at[b], gsem).wait()
            pltpu.make_async_copy(table.at[idxb.at[1, b]], R1.at[b], gsem).wait()

        for q in range(_CH // 16):
            prow = iota + q * 16
            for d in range(3):
                dd = jnp.full((16,), d, jnp.int32)
                v = plsc.load_gather(ptsb, [prow, dd])
                plsc.store_scatter(O, [prow, dd], v)

            @pl.loop(0, _NB)
            def _b(b):
                w1v = w1b[b, pl.ds(q * 16, 16)]
                yv = yb[b, pl.ds(q * 16, 16)]
                bs = jnp.full((16,), b, jnp.int32)
                for c in range(_C):
                    cs = jnp.full((16,), c, jnp.int32)
                    v0 = plsc.load_gather(R0, [bs, prow, cs])
                    v1 = plsc.load_gather(R1, [bs, prow, cs])
                    o = v0 + w1v * (v1 - v0) + yv
                    col = jnp.full((16,), 3 + c * _NB, jnp.int32) + bs
                    plsc.store_scatter(O, [prow, col], o)

        pltpu.sync_copy(O, out.at[pl.ds(n0, _CH), :])


def _sc(table, idxI, w1T, yT, pts):
    mesh = plsc.VectorSubcoreMesh(
        core_axis_name="c", subcore_axis_name="s", num_cores=2, num_subcores=16
    )
    return pl.kernel(
        _sc_body,
        out_type=jax.ShapeDtypeStruct((_N, _OUT), jnp.float32),
        mesh=mesh,
        compiler_params=pltpu.CompilerParams(
            needs_layout_passes=False, use_tc_tiling_on_sc=False
        ),
        scratch_types=[
            pltpu.VMEM((2, _NB, _CH), jnp.int32),
            pltpu.VMEM((_NB, _CH), jnp.float32),
            pltpu.VMEM((_NB, _CH), jnp.float32),
            pltpu.VMEM((_CH, 3), jnp.float32),
            pltpu.VMEM((_NB, _CH, _C), jnp.float32),
            pltpu.VMEM((_NB, _CH, _C), jnp.float32),
            pltpu.VMEM((_CH, _OUT), jnp.float32),
            pltpu.SemaphoreType.DMA,
        ],
    )(table, idxI, w1T, yT, pts)


def kernel(points, features, scales):
    cv = features[..., 0]
    table = _table(cv)
    idxI, w1T, yT = _prep(scales, points.T)
    return _sc(table, idxI, w1T, yT, points)


# speedup vs baseline: 2.3148x; 1.3556x over previous
import jax
import jax.numpy as jnp
from jax import lax
from jax.experimental import pallas as pl
from jax.experimental.pallas import tpu as pltpu
from jax.experimental.pallas import tpu_sc as plsc

_N = 131072
_NB = 36
_C = 8
_H = 65536
_V = _NB * _H
_OUT = 3 + _NB * _C

_NW = 32
_NPW = _N // _NW
_CH = 64
_NCH = _NPW // _CH


_PNB = 2048


def _prep_body(scales_ref, pts_ref, idx_ref, w1_ref):
    p3 = pts_ref[...]
    encs = []
    for f in range(6):
        s = scales_ref[f]
        fp = p3 * s
        encs.append(jnp.sin(fp))
        encs.append(jnp.cos(fp))
    enc = jnp.concatenate(encs, axis=0)
    iy = (enc + 1.0) * 32767.5
    iy0 = jnp.floor(iy)
    w1_ref[...] = iy - iy0
    i0 = jnp.clip(iy0, 0.0, 65535.0).astype(jnp.int32)
    boff = lax.broadcasted_iota(jnp.int32, (_NB, _PNB), 0) * _H
    idx_ref[...] = i0 + boff


def _prep(scales, points_t):
    return pl.pallas_call(
        _prep_body,
        grid=(_N // _PNB,),
        in_specs=[
            pl.BlockSpec(memory_space=pltpu.SMEM),
            pl.BlockSpec((3, _PNB), lambda i: (0, i)),
        ],
        out_specs=[
            pl.BlockSpec((_NB, _PNB), lambda i: (0, i)),
            pl.BlockSpec((_NB, _PNB), lambda i: (0, i)),
        ],
        out_shape=[
            jax.ShapeDtypeStruct((_NB, _N), jnp.int32),
            jax.ShapeDtypeStruct((_NB, _N), jnp.float32),
        ],
    )(scales, points_t)


_TW = 2048
_NU = _NB * (_H // _TW)
_UPW = _NU // _NW
_NWIN = _H // _TW


def _tr_body(feat, table, xb, ob, isem, osem):
    wid = lax.axis_index("s") * 2 + lax.axis_index("c")
    iota = lax.iota(jnp.int32, 16)
    u0 = wid * _UPW
    cvec = jnp.bitwise_and(iota, 7)
    hi8 = lax.shift_right_logical(iota, 3)
    lo8 = iota < 8

    def src(u):
        return feat.at[u // _NWIN, :, pl.ds((u % _NWIN) * _TW, _TW)]

    def srcx(u):
        return feat.at[u // _NWIN, :, pl.ds((u % _NWIN) * _TW + _TW, 1)]

    def dst(u):
        return table.at[pl.ds((u // _NWIN) * _H + (u % _NWIN) * _TW, _TW), :]

    def fire_in(u, slot):
        pltpu.async_copy(src(u), xb.at[slot, :, pl.ds(0, _TW)], isem.at[slot])

        @pl.when(u % _NWIN < _NWIN - 1)
        def _x():
            pltpu.async_copy(srcx(u), xb.at[slot, :, pl.ds(_TW, 1)], isem.at[slot])

    def wait_in(u, slot):
        pltpu.make_async_copy(src(u), xb.at[slot, :, pl.ds(0, _TW)], isem.at[slot]).wait()

        @pl.when(u % _NWIN < _NWIN - 1)
        def _x():
            pltpu.make_async_copy(
                srcx(u), xb.at[slot, :, pl.ds(_TW, 1)], isem.at[slot]
            ).wait()

    fire_in(u0, 0)

    @pl.loop(0, _UPW)
    def _unit(j):
        u = u0 + j
        slot = lax.rem(j, 2)
        wait_in(u, slot)

        @pl.when(j + 1 < _UPW)
        def _next():
            fire_in(u + 1, 1 - slot)

        @pl.when(j >= 2)
        def _wo():
            pltpu.make_async_copy(ob.at[slot], dst(u), osem.at[slot]).wait()

        xf = xb.at[slot]
        of = ob.at[slot]

        @pl.when(u % _NWIN == _NWIN - 1)
        def _dup():
            v = plsc.load_gather(
                xf, [cvec, jnp.full((16,), _TW - 1, jnp.int32)], mask=lo8
            )
            plsc.store_scatter(
                xf, [cvec, jnp.full((16,), _TW, jnp.int32)], v, mask=lo8
            )

        @pl.loop(0, _TW // 2, unroll=4)
        def _col(p):
            col = hi8 + p * 2
            v = plsc.load_gather(xf, [cvec, col])
            plsc.store_scatter(of, [col, cvec], v)
            vs = plsc.load_gather(xf, [cvec, col + 1])
            plsc.store_scatter(of, [col, cvec + 8], vs)

        pltpu.async_copy(ob.at[slot], dst(u), osem.at[slot])

    pltpu.make_async_copy(ob.at[0], dst(u0), osem.at[0]).wait()
    pltpu.make_async_copy(ob.at[1], dst(u0), osem.at[1]).wait()


def _table(feat):
    mesh = plsc.VectorSubcoreMesh(
        core_axis_name="c", subcore_axis_name="s", num_cores=2, num_subcores=16
    )
    return pl.kernel(
        _tr_body,
        out_type=jax.ShapeDtypeStruct((_V, 2 * _C), jnp.float32),
        mesh=mesh,
        compiler_params=pltpu.CompilerParams(
            needs_layout_passes=False, use_tc_tiling_on_sc=False
        ),
        scratch_types=[
            pltpu.VMEM((2, _C, _TW + 1), jnp.float32),
            pltpu.VMEM((2, _TW, 2 * _C), jnp.float32),
            pltpu.SemaphoreType.DMA((2,)),
            pltpu.SemaphoreType.DMA((2,)),
        ],
    )(feat)


def _sc_body(table, idxT, w1T, pts, out, idxb, w1b, ptsb, R, O, isem, gsem, osem):
    wid = lax.axis_index("s") * 2 + lax.axis_index("c")
    base = wid * _NPW
    iota = lax.iota(jnp.int32, 16)

    def in_refs(j, s):
        n0 = base + j * _CH
        return (
            (idxT.at[:, pl.ds(n0, _CH)], idxb.at[s]),
            (w1T.at[:, pl.ds(n0, _CH)], w1b.at[s]),
            (pts.at[pl.ds(n0, _CH), :], ptsb.at[s]),
        )

    def fire_in(j, s):
        for src, dstb in in_refs(j, s):
            pltpu.async_copy(src, dstb, isem.at[s])

    def wait_in(j, s):
        for src, dstb in in_refs(j, s):
            pltpu.make_async_copy(src, dstb, isem.at[s]).wait()

    def fire_gathers(s):
        @pl.loop(0, _NB)
        def _f(b):
            pltpu.async_copy(table.at[idxb.at[s, b]], R.at[s, b], gsem.at[s])

    def wait_gathers(s):
        @pl.loop(0, _NB)
        def _d(b):
            pltpu.make_async_copy(table.at[idxb.at[s, b]], R.at[s, b], gsem.at[s]).wait()

    fire_in(0, 0)
    wait_in(0, 0)
    fire_gathers(0)
    fire_in(1, 1)

    @pl.loop(0, _NCH)
    def _chunk(j):
        s = lax.rem(j, 2)
        ns = 1 - s
        n0 = base + j * _CH
        wait_gathers(s)

        @pl.when(j + 1 < _NCH)
        def _pg():
            wait_in(j + 1, ns)
            fire_gathers(ns)

        @pl.when(j >= 2)
        def _wo():
            pltpu.make_async_copy(O.at[s], out.at[pl.ds(n0, _CH), :], osem.at[s]).wait()

        for q in range(_CH // 16):
            prow = iota + q * 16
            for d in range(3):
                dd = jnp.full((16,), d, jnp.int32)
                v = plsc.load_gather(ptsb.at[s], [prow, dd])
                plsc.store_scatter(O.at[s], [prow, dd], v)

            @pl.loop(0, _NB)
            def _b(b):
                w1v = w1b[s, b, pl.ds(q * 16, 16)]
                g0v = idxb[s, b, pl.ds(q * 16, 16)]
                yv = ((g0v - b * _H).astype(jnp.float32) + w1v) * (1.0 / 32767.5) - 1.0
                bs = jnp.full((16,), b, jnp.int32)
                for c in range(_C):
                    cs = jnp.full((16,), c, jnp.int32)
                    v0 = plsc.load_gather(R.at[s], [bs, prow, cs])
                    v1 = plsc.load_gather(R.at[s], [bs, prow, cs + 8])
                    o = v0 + w1v * (v1 - v0) + yv
                    col = jnp.full((16,), 3 + c * _NB, jnp.int32) + bs
                    plsc.store_scatter(O.at[s], [prow, col], o)

        pltpu.async_copy(O.at[s], out.at[pl.ds(n0, _CH), :], osem.at[s])

        @pl.when(j + 2 < _NCH)
        def _ni():
            fire_in(j + 2, s)

    pltpu.make_async_copy(O.at[0], out.at[pl.ds(base, _CH), :], osem.at[0]).wait()
    pltpu.make_async_copy(O.at[1], out.at[pl.ds(base, _CH), :], osem.at[1]).wait()


def _sc(table, idxT, w1T, pts):
    mesh = plsc.VectorSubcoreMesh(
        core_axis_name="c", subcore_axis_name="s", num_cores=2, num_subcores=16
    )
    return pl.kernel(
        _sc_body,
        out_type=jax.ShapeDtypeStruct((_N, _OUT), jnp.float32),
        mesh=mesh,
        compiler_params=pltpu.CompilerParams(
            needs_layout_passes=False, use_tc_tiling_on_sc=False
        ),
        scratch_types=[
            pltpu.VMEM((2, _NB, _CH), jnp.int32),
            pltpu.VMEM((2, _NB, _CH), jnp.float32),
            pltpu.VMEM((2, _CH, 3), jnp.float32),
            pltpu.VMEM((2, _NB, _CH, 2 * _C), jnp.float32),
            pltpu.VMEM((2, _CH, _OUT), jnp.float32),
            pltpu.SemaphoreType.DMA((2,)),
            pltpu.SemaphoreType.DMA((2,)),
            pltpu.SemaphoreType.DMA((2,)),
        ],
    )(table, idxT, w1T, pts)


def kernel(points, features, scales):
    cv = features.reshape(_NB, _C, _H)
    table = _table(cv)
    idxT, w1T = _prep(scales, points.T)
    return _sc(table, idxT, w1T, points)


# speedup vs baseline: 3.1521x; 1.3617x over previous
import jax
import jax.numpy as jnp
from jax import lax
from jax.experimental import pallas as pl
from jax.experimental.pallas import tpu as pltpu
from jax.experimental.pallas import tpu_sc as plsc

_N = 131072
_NB = 36
_C = 8
_H = 65536
_V = _NB * _H
_VR = _V // 2
_OUT = 3 + _NB * _C

_NW = 32
_NPW = _N // _NW
_CH = 64
_NCH = _NPW // _CH


_PNB = 2048


def _prep_body(scales_ref, pts_ref, idx_ref, w1_ref):
    p3 = pts_ref[...]
    encs = []
    for f in range(6):
        s = scales_ref[f]
        fp = p3 * s
        encs.append(jnp.sin(fp))
        encs.append(jnp.cos(fp))
    enc = jnp.concatenate(encs, axis=0)
    iy = (enc + 1.0) * 32767.5
    iy0 = jnp.floor(iy)
    w1_ref[...] = iy - iy0
    i0 = jnp.clip(iy0, 0.0, 65535.0).astype(jnp.int32)
    boff = lax.broadcasted_iota(jnp.int32, (_NB, _PNB), 0) * _H
    idx_ref[...] = i0 + boff


def _prep(scales, points_t):
    return pl.pallas_call(
        _prep_body,
        grid=(_N // _PNB,),
        in_specs=[
            pl.BlockSpec(memory_space=pltpu.SMEM),
            pl.BlockSpec((3, _PNB), lambda i: (0, i)),
        ],
        out_specs=[
            pl.BlockSpec((_NB, _PNB), lambda i: (0, i)),
            pl.BlockSpec((_NB, _PNB), lambda i: (0, i)),
        ],
        out_shape=[
            jax.ShapeDtypeStruct((_NB, _N), jnp.int32),
            jax.ShapeDtypeStruct((_NB, _N), jnp.float32),
        ],
    )(scales, points_t)


_TW = 2048
_NU = _NB * (_H // _TW)
_UPW = _NU // _NW
_NWIN = _H // _TW
_TR = _TW // 2


def _tr_body(feat, table, xb, ob, isem, osem):
    wid = lax.axis_index("s") * 2 + lax.axis_index("c")
    iota = lax.iota(jnp.int32, 16)
    u0 = wid * _UPW
    lo8 = iota < 8
    cv8 = jnp.bitwise_and(iota, 7)

    def src(u):
        return feat.at[u // _NWIN, :, pl.ds((u % _NWIN) * _TW, _TW)]

    def srcx(u):
        return feat.at[u // _NWIN, :, pl.ds((u % _NWIN) * _TW + _TW, 2)]

    def dst(u):
        return table.at[pl.ds((u // _NWIN) * (_H // 2) + (u % _NWIN) * _TR, _TR), :]

    def fire_in(u, slot):
        pltpu.async_copy(src(u), xb.at[slot, :, pl.ds(0, _TW)], isem.at[slot])

        @pl.when(u % _NWIN < _NWIN - 1)
        def _x():
            pltpu.async_copy(srcx(u), xb.at[slot, :, pl.ds(_TW, 2)], isem.at[slot])

    def wait_in(u, slot):
        pltpu.make_async_copy(src(u), xb.at[slot, :, pl.ds(0, _TW)], isem.at[slot]).wait()

        @pl.when(u % _NWIN < _NWIN - 1)
        def _x():
            pltpu.make_async_copy(
                srcx(u), xb.at[slot, :, pl.ds(_TW, 2)], isem.at[slot]
            ).wait()

    fire_in(u0, 0)

    @pl.loop(0, _UPW)
    def _unit(j):
        u = u0 + j
        slot = lax.rem(j, 2)
        wait_in(u, slot)

        @pl.when(j + 1 < _UPW)
        def _next():
            fire_in(u + 1, 1 - slot)

        @pl.when(j >= 2)
        def _wo():
            pltpu.make_async_copy(ob.at[slot], dst(u), osem.at[slot]).wait()

        xf = xb.at[slot]
        of = ob.at[slot]

        @pl.when(u % _NWIN == _NWIN - 1)
        def _dup():
            v = plsc.load_gather(
                xf, [cv8, jnp.full((16,), _TW - 1, jnp.int32)], mask=lo8
            )
            plsc.store_scatter(xf, [cv8, jnp.full((16,), _TW, jnp.int32)], v, mask=lo8)
            plsc.store_scatter(
                xf, [cv8, jnp.full((16,), _TW + 1, jnp.int32)], v, mask=lo8
            )

        @pl.loop(0, _TR // 16)
        def _grp(eg):
            evec = iota + eg * 16
            col2 = evec * 2
            for t in range(4):
                for k in range(4):
                    va = plsc.load_gather(xf, [jnp.full((16,), 2 * k, jnp.int32), col2 + t])
                    vb = plsc.load_gather(
                        xf, [jnp.full((16,), 2 * k + 1, jnp.int32), col2 + t]
                    )
                    wd = plsc.bitcast(
                        plsc.pack(va, vb, format=plsc.PackFormat.INTERLEAVED), jnp.int32
                    )
                    plsc.store_scatter(
                        of, [evec, jnp.full((16,), t * 4 + k, jnp.int32)], wd
                    )

        pltpu.async_copy(ob.at[slot], dst(u), osem.at[slot])

    pltpu.make_async_copy(ob.at[0], dst(u0), osem.at[0]).wait()
    pltpu.make_async_copy(ob.at[1], dst(u0), osem.at[1]).wait()


def _table(feat):
    mesh = plsc.VectorSubcoreMesh(
        core_axis_name="c", subcore_axis_name="s", num_cores=2, num_subcores=16
    )
    return pl.kernel(
        _tr_body,
        out_type=jax.ShapeDtypeStruct((_VR, 16), jnp.int32),
        mesh=mesh,
        compiler_params=pltpu.CompilerParams(
            needs_layout_passes=False, use_tc_tiling_on_sc=False
        ),
        scratch_types=[
            pltpu.VMEM((2, _C, _TW + 2), jnp.float32),
            pltpu.VMEM((2, _TR, 16), jnp.int32),
            pltpu.SemaphoreType.DMA((2,)),
            pltpu.SemaphoreType.DMA((2,)),
        ],
    )(feat)


def _sc_body(table, idxT, w1T, pts, out, idxb, Ab, w1b, ptsb, R, O, isem, gsem, osem):
    wid = lax.axis_index("s") * 2 + lax.axis_index("c")
    base = wid * _NPW
    iota = lax.iota(jnp.int32, 16)

    def in_refs(j, s):
        n0 = base + j * _CH
        return (
            (idxT.at[:, pl.ds(n0, _CH)], idxb.at[s]),
            (w1T.at[:, pl.ds(n0, _CH)], w1b.at[s]),
            (pts.at[pl.ds(n0, _CH), :], ptsb.at[s]),
        )

    def fire_in(j, s):
        for src, dstb in in_refs(j, s):
            pltpu.async_copy(src, dstb, isem.at[s])

    def wait_in(j, s):
        for src, dstb in in_refs(j, s):
            pltpu.make_async_copy(src, dstb, isem.at[s]).wait()

    def derive_idx(s):
        @pl.loop(0, _NB)
        def _b(b):
            for qq in range(_CH // 16):
                g = idxb[s, b, pl.ds(qq * 16, 16)]
                Ab[s, b, pl.ds(qq * 16, 16)] = lax.shift_right_logical(g, 1)

    def fire_gathers(s):
        @pl.loop(0, _NB)
        def _f(b):
            pltpu.async_copy(table.at[Ab.at[s, b]], R.at[s, b], gsem.at[s])

    def wait_gathers(s):
        @pl.loop(0, _NB)
        def _d(b):
            pltpu.make_async_copy(table.at[Ab.at[s, b]], R.at[s, b], gsem.at[s]).wait()

    fire_in(0, 0)
    wait_in(0, 0)
    derive_idx(0)
    fire_gathers(0)
    fire_in(1, 1)

    @pl.loop(0, _NCH)
    def _chunk(j):
        s = lax.rem(j, 2)
        ns = 1 - s
        n0 = base + j * _CH
        wait_gathers(s)

        @pl.when(j + 1 < _NCH)
        def _pg():
            wait_in(j + 1, ns)
            derive_idx(ns)
            fire_gathers(ns)

        @pl.when(j >= 2)
        def _wo():
            pltpu.make_async_copy(O.at[s], out.at[pl.ds(n0, _CH), :], osem.at[s]).wait()

        for q in range(_CH // 16):
            prow = iota + q * 16
            for d in range(3):
                dd = jnp.full((16,), d, jnp.int32)
                v = plsc.load_gather(ptsb.at[s], [prow, dd])
                plsc.store_scatter(O.at[s], [prow, dd], v)

            @pl.loop(0, _NB)
            def _b(b):
                w1v = w1b[s, b, pl.ds(q * 16, 16)]
                g0v = idxb[s, b, pl.ds(q * 16, 16)]
                yv = ((g0v - b * _H).astype(jnp.float32) + w1v) * (1.0 / 32767.5) - 1.0
                par4 = lax.shift_left(jnp.bitwise_and(g0v, 1), 2)
                bs = jnp.full((16,), b, jnp.int32)
                for k in range(4):
                    w0 = plsc.load_gather(R.at[s], [bs, prow, par4 + k])
                    w1d = plsc.load_gather(R.at[s], [bs, prow, par4 + (k + 4)])
                    a0, b0 = plsc.unpack(
                        plsc.bitcast(w0, jnp.bfloat16), format=plsc.PackFormat.INTERLEAVED
                    )
                    a1, b1 = plsc.unpack(
                        plsc.bitcast(w1d, jnp.bfloat16), format=plsc.PackFormat.INTERLEAVED
                    )
                    a0 = a0.astype(jnp.float32)
                    b0 = b0.astype(jnp.float32)
                    a1 = a1.astype(jnp.float32)
                    b1 = b1.astype(jnp.float32)
                    oa = a0 + w1v * (a1 - a0) + yv
                    obv = b0 + w1v * (b1 - b0) + yv
                    ca = jnp.full((16,), 3 + (2 * k) * _NB, jnp.int32) + bs
                    cb = jnp.full((16,), 3 + (2 * k + 1) * _NB, jnp.int32) + bs
                    plsc.store_scatter(O.at[s], [prow, ca], oa)
                    plsc.store_scatter(O.at[s], [prow, cb], obv)

        pltpu.async_copy(O.at[s], out.at[pl.ds(n0, _CH), :], osem.at[s])

        @pl.when(j + 2 < _NCH)
        def _ni():
            fire_in(j + 2, s)

    pltpu.make_async_copy(O.at[0], out.at[pl.ds(base, _CH), :], osem.at[0]).wait()
    pltpu.make_async_copy(O.at[1], out.at[pl.ds(base, _CH), :], osem.at[1]).wait()


def _sc(table, idxT, w1T, pts):
    mesh = plsc.VectorSubcoreMesh(
        core_axis_name="c", subcore_axis_name="s", num_cores=2, num_subcores=16
    )
    return pl.kernel(
        _sc_body,
        out_type=jax.ShapeDtypeStruct((_N, _OUT), jnp.float32),
        mesh=mesh,
        compiler_params=pltpu.CompilerParams(
            needs_layout_passes=False, use_tc_tiling_on_sc=False
        ),
        scratch_types=[
            pltpu.VMEM((2, _NB, _CH), jnp.int32),
            pltpu.VMEM((2, _NB, _CH), jnp.int32),
            pltpu.VMEM((2, _NB, _CH), jnp.float32),
            pltpu.VMEM((2, _CH, 3), jnp.float32),
            pltpu.VMEM((2, _NB, _CH, 16), jnp.int32),
            pltpu.VMEM((2, _CH, _OUT), jnp.float32),
            pltpu.SemaphoreType.DMA((2,)),
            pltpu.SemaphoreType.DMA((2,)),
            pltpu.SemaphoreType.DMA((2,)),
        ],
    )(table, idxT, w1T, pts)


def kernel(points, features, scales):
    cv = features.reshape(_NB, _C, _H)
    table = _table(cv)
    idxT, w1T = _prep(scales, points.T)
    return _sc(table, idxT, w1T, points)
